# baseline (speedup 1.0000x reference)
import functools

import jax
import jax.numpy as jnp
from jax import lax
from jax.experimental import pallas as pl
from jax.experimental.pallas import tpu as pltpu
from jax.experimental.pallas import tpu_sc as plsc

D_MODEL = 1024
SEQ_LEN = 2048
BATCH = 4

_INFO = plsc.get_sparse_core_info()
_NC, _NS, _L = _INFO.num_cores, _INFO.num_subcores, _INFO.num_lanes
_NW = _NC * _NS

ROWS_PER_W = SEQ_LEN // _NW
STEP_ROWS = 2
N_STEPS = ROWS_PER_W // STEP_ROWS
NSLOTS = 12
UNROLL = 2
GROUPS = D_MODEL // _L

_mesh = plsc.VectorSubcoreMesh(core_axis_name="c", subcore_axis_name="s")


@functools.partial(
    pl.kernel,
    mesh=_mesh,
    out_type=jax.ShapeDtypeStruct((BATCH, SEQ_LEN, D_MODEL), jnp.float32),
    scratch_types=(
        [pltpu.VMEM((STEP_ROWS, D_MODEL), jnp.float32) for _ in range(NSLOTS * BATCH)]
        + [pltpu.VMEM((STEP_ROWS, D_MODEL), jnp.float32) for _ in range(NSLOTS)]
        + [pltpu.SemaphoreType.DMA for _ in range(2 * NSLOTS)]
    ),
)
def _pe_add(x_hbm, pe_hbm, out_hbm, *scratch):
    xbufs = tuple(
        tuple(scratch[s * BATCH + b] for b in range(BATCH)) for s in range(NSLOTS)
    )
    pebufs = tuple(scratch[NSLOTS * BATCH + s] for s in range(NSLOTS))
    in_sems = tuple(scratch[NSLOTS * (BATCH + 1) + s] for s in range(NSLOTS))
    out_sems = tuple(scratch[NSLOTS * (BATCH + 2) + s] for s in range(NSLOTS))

    wid = lax.axis_index("s") * _NC + lax.axis_index("c")
    band0 = wid * ROWS_PER_W

    def row0(k):
        return band0 + k * STEP_ROWS

    def start_in(k):
        s = k % NSLOTS
        r0 = row0(k)
        pltpu.async_copy(pe_hbm.at[pl.ds(r0, STEP_ROWS)], pebufs[s], in_sems[s])
        for b in range(BATCH):
            pltpu.async_copy(
                x_hbm.at[b, pl.ds(r0, STEP_ROWS)], xbufs[s][b], in_sems[s]
            )

    def wait_in(k):
        s = k % NSLOTS
        r0 = row0(k)
        pltpu.make_async_copy(
            pe_hbm.at[pl.ds(r0, STEP_ROWS)], pebufs[s], in_sems[s]
        ).wait()
        for b in range(BATCH):
            pltpu.make_async_copy(
                x_hbm.at[b, pl.ds(r0, STEP_ROWS)], xbufs[s][b], in_sems[s]
            ).wait()

    def start_out(k):
        s = k % NSLOTS
        r0 = row0(k)
        for b in range(BATCH):
            pltpu.async_copy(
                xbufs[s][b], out_hbm.at[b, pl.ds(r0, STEP_ROWS)], out_sems[s]
            )

    def wait_out(k):
        s = k % NSLOTS
        r0 = row0(k)
        for b in range(BATCH):
            pltpu.make_async_copy(
                xbufs[s][b], out_hbm.at[b, pl.ds(r0, STEP_ROWS)], out_sems[s]
            ).wait()

    for k in range(min(N_STEPS, NSLOTS - 1)):
        start_in(k)

    for k in range(N_STEPS):
        s = k % NSLOTS
        wait_in(k)
        xb = xbufs[s]
        peb = pebufs[s]

        for r in range(STEP_ROWS):

            def body(g, _, xb=xb, peb=peb, r=r):
                col = g * (_L * UNROLL)
                for u in range(UNROLL):
                    o = col + u * _L
                    p = peb[r, pl.ds(o, _L)]
                    for b in range(BATCH):
                        xb[b][r, pl.ds(o, _L)] = xb[b][r, pl.ds(o, _L)] + p
                return 0

            lax.fori_loop(0, GROUPS // UNROLL, body, 0)

        start_out(k)
        nxt = k + NSLOTS - 1
        if nxt < N_STEPS:
            if nxt >= NSLOTS:
                wait_out(nxt - NSLOTS)
            start_in(nxt)

    for k in range(max(0, N_STEPS - NSLOTS), N_STEPS):
        wait_out(k)


def kernel(x, pe):
    return _pe_add(x, pe)
